# baseline (speedup 1.0000x reference)
import functools

import jax
import jax.numpy as jnp
from jax import lax
from jax.experimental import pallas as pl
from jax.experimental.pallas import tpu as pltpu
from jax.experimental.pallas import tpu_sc as plsc

N = 10000
E = 320000
D = 128

NC = 2
NS = 16
NW = NC * NS

NPAD = 10112
RPT = NPAD // NS
EPAD = 327680
EW = EPAD // NW
KB = 128
CH = EW // KB

_mesh = plsc.VectorSubcoreMesh(core_axis_name="c", subcore_axis_name="s")

f32 = jnp.float32
i32 = jnp.int32


def _wid():
    return lax.axis_index("s") * NC + lax.axis_index("c")


@functools.partial(
    pl.kernel,
    out_type=jax.ShapeDtypeStruct((NC, NPAD, D), f32),
    mesh=_mesh,
    scratch_types=[
        pltpu.VMEM((CH, KB), i32),
        pltpu.VMEM((KB, D), f32),
        pltpu.VMEM_SHARED((NPAD, D), f32),
    ],
)
def _sc_deg(dst_hbm, zeros16_hbm, ones16_hbm, out_hbm, idxd, ones_v, shared):
    c = lax.axis_index("c")
    s = lax.axis_index("s")
    w = _wid()
    pltpu.sync_copy(zeros16_hbm.at[pl.ds(s * RPT, RPT)],
                    shared.at[pl.ds(s * RPT, RPT)])
    pltpu.sync_copy(ones16_hbm, ones_v)
    pltpu.sync_copy(dst_hbm.at[pl.ds(w * CH, CH)], idxd)
    plsc.subcore_barrier()

    def body(r, _):
        pltpu.sync_copy(ones_v, shared.at[idxd.at[r]], add=True)
        return 0

    lax.fori_loop(0, CH, body, 0)
    plsc.subcore_barrier()
    pltpu.sync_copy(shared.at[pl.ds(s * RPT, RPT)],
                    out_hbm.at[c, pl.ds(s * RPT, RPT)])


TCH = EPAD // KB
CONV_F = 80
CONV_S = TCH // NS - CONV_F
EDGE_F = 80
EDGE_S = TCH // NS - EDGE_F


@functools.partial(
    pl.kernel,
    out_type=jax.ShapeDtypeStruct((NC, NPAD, D), f32),
    mesh=_mesh,
    scratch_types=[
        pltpu.VMEM((3, KB), i32),
        pltpu.VMEM((3, KB), i32),
        pltpu.VMEM((KB, D), f32),
        pltpu.VMEM((KB, D), f32),
        pltpu.VMEM((KB, D), f32),
        pltpu.VMEM_SHARED((NPAD, D), f32),
        pltpu.SemaphoreType.DMA,
        pltpu.SemaphoreType.DMA,
        pltpu.SemaphoreType.DMA,
        pltpu.SemaphoreType.DMA,
        pltpu.SemaphoreType.DMA,
        pltpu.SemaphoreType.DMA,
        pltpu.SemaphoreType.DMA,
    ],
)
def _sc_conv(hs_hbm, src_hbm, dst_hbm, zeros_hbm, out_hbm,
             sring, dring, rows0, rows1, rows2, shared,
             gsem, ss0, ss1, ss2, sd0, sd1, sd2):
    c = lax.axis_index("c")
    s = lax.axis_index("s")
    myn = jnp.where(c == 1, CONV_F, CONV_S)
    base = jnp.where(c == 1, s * CONV_F, NS * CONV_F + s * CONV_S)
    pltpu.sync_copy(zeros_hbm.at[pl.ds(s * RPT, RPT)],
                    shared.at[pl.ds(s * RPT, RPT)])
    plsc.subcore_barrier()

    allrows = (rows0, rows1, rows2)
    sss = (ss0, ss1, ss2)
    sds = (sd0, sd1, sd2)
    for j in range(3):
        pltpu.async_copy(src_hbm.at[base + j], sring.at[j], sss[j])
        pltpu.async_copy(dst_hbm.at[base + j], dring.at[j], sds[j])
    for j in range(3):
        pltpu.make_async_copy(src_hbm.at[base], sring.at[j], sss[j]).wait()
        pltpu.async_copy(hs_hbm.at[sring.at[j]], allrows[j], gsem)

    def half(r, rows, sslot, dslot, ssem, dsem):
        r3 = r + 3

        @pl.when(r < myn)
        def _():
            pltpu.make_async_copy(hs_hbm.at[sslot], rows, gsem).wait()

            @pl.when(r3 < myn)
            def _():
                pltpu.async_copy(src_hbm.at[base + r3], sslot, ssem)

            pltpu.make_async_copy(dst_hbm.at[base], dslot, dsem).wait()
            pltpu.sync_copy(rows, shared.at[dslot], add=True)

            @pl.when(r3 < myn)
            def _():
                pltpu.async_copy(dst_hbm.at[base + r3], dslot, dsem)
                pltpu.make_async_copy(src_hbm.at[base], sslot, ssem).wait()
                pltpu.async_copy(hs_hbm.at[sslot], rows, gsem)

    def body(i, _):
        for j in range(3):
            half(3 * i + j, allrows[j], sring.at[j], dring.at[j],
                 sss[j], sds[j])
        return 0

    lax.fori_loop(0, (myn + 2) // 3, body, 0)
    plsc.subcore_barrier()
    pltpu.sync_copy(shared.at[pl.ds(s * RPT, RPT)],
                    out_hbm.at[c, pl.ds(s * RPT, RPT)])


_G = 8


@functools.partial(
    pl.kernel,
    out_type=jax.ShapeDtypeStruct((EPAD * 16,), f32),
    mesh=_mesh,
    scratch_types=[
        pltpu.VMEM((2, KB), i32),
        pltpu.VMEM((2, KB), i32),
        pltpu.VMEM((KB, D), f32),
        pltpu.VMEM((KB, D), f32),
        pltpu.VMEM((KB, D), f32),
        pltpu.VMEM((KB, D), f32),
        pltpu.VMEM((_G * KB * 16,), f32),
        pltpu.VMEM((D,), f32),
        pltpu.SemaphoreType.DMA,
        pltpu.SemaphoreType.DMA,
        pltpu.SemaphoreType.DMA,
        pltpu.SemaphoreType.DMA,
    ],
)
def _sc_edge(a_hbm, b_hbm, row_hbm, col_hbm, w1_hbm, out_hbm,
             rring, cring, a0, b0, a1, b1, pbuf, w1_v, sema, semb, sr0, sr1):
    c = lax.axis_index("c")
    s = lax.axis_index("s")
    myn = jnp.where(c == 1, EDGE_F, EDGE_S)
    base = jnp.where(c == 1, s * EDGE_F, NS * EDGE_F + s * EDGE_S)
    pltpu.sync_copy(w1_hbm, w1_v)
    wv = [w1_v[pl.ds(16 * k, 16)] for k in range(D // 16)]

    def load_idx(r, slot, sem):
        pltpu.async_copy(row_hbm.at[base + r], rring.at[slot], sem)
        pltpu.async_copy(col_hbm.at[base + r], cring.at[slot], sem)

    def wait_idx(slot, sem):
        pltpu.make_async_copy(row_hbm.at[base], rring.at[slot], sem).wait()
        pltpu.make_async_copy(col_hbm.at[base], cring.at[slot], sem).wait()

    load_idx(0, 0, sr0)
    load_idx(1, 1, sr1)
    wait_idx(0, sr0)
    pltpu.async_copy(a_hbm.at[rring.at[0]], a0, sema)
    pltpu.async_copy(b_hbm.at[cring.at[0]], b0, semb)
    wait_idx(1, sr1)
    pltpu.async_copy(a_hbm.at[rring.at[1]], a1, sema)
    pltpu.async_copy(b_hbm.at[cring.at[1]], b1, semb)

    def outer(g, _):
        rbase = g * _G
        for t in range(_G):
            r = rbase + t
            r2 = r + 2
            slot = t % 2
            cur_a, cur_b = (a0, b0) if slot == 0 else (a1, b1)
            ssem = sr0 if slot == 0 else sr1
            pltpu.make_async_copy(a_hbm.at[rring.at[slot]], cur_a, sema).wait()
            pltpu.make_async_copy(b_hbm.at[cring.at[slot]], cur_b, semb).wait()

            @pl.when(r2 < myn)
            def _():
                load_idx(r2, slot, ssem)

            def edge(e, _):
                acc = jnp.zeros((16,), f32)
                for k in range(D // 16):
                    va = cur_a[e, pl.ds(16 * k, 16)]
                    vb = cur_b[e, pl.ds(16 * k, 16)]
                    acc = acc + jnp.maximum(va + vb, 0.0) * wv[k]
                pbuf[pl.ds((t * KB + e) * 16, 16)] = acc
                return 0

            lax.fori_loop(0, KB, edge, 0)

            @pl.when(r2 < myn)
            def _():
                wait_idx(slot, ssem)
                pltpu.async_copy(a_hbm.at[rring.at[slot]], cur_a, sema)
                pltpu.async_copy(b_hbm.at[cring.at[slot]], cur_b, semb)

        pltpu.sync_copy(
            pbuf, out_hbm.at[pl.ds((base + rbase) * KB * 16, _G * KB * 16)])
        return 0

    lax.fori_loop(0, myn // _G, outer, 0)


def _dinv_from_dp(dp):
    deg = dp[0, :N, 0:1] + dp[1, :N, 0:1] + 1.0
    return lax.rsqrt(deg)


def _tc_a_body(x_ref, w_ref, dp_ref, h_ref, hsp_ref):
    dv = _dinv_from_dp(dp_ref[...])
    h = jnp.dot(x_ref[...], w_ref[...], preferred_element_type=f32)
    h_ref[...] = h
    hsp_ref[0:N, :] = dv * h
    hsp_ref[N:NPAD, :] = jnp.zeros((NPAD - N, D), f32)


def _tc_a(x, w0, dp):
    return pl.pallas_call(
        _tc_a_body,
        out_shape=[
            jax.ShapeDtypeStruct((N, D), f32),
            jax.ShapeDtypeStruct((NPAD, D), f32),
        ],
    )(x, w0, dp)


def _tc_b_body(accp_ref, hprev_ref, dp_ref, b_ref, g_ref, be_ref, w_ref,
               h_ref, hsp_ref):
    dv = _dinv_from_dp(dp_ref[...])
    acc = accp_ref[0, 0:N, :] + accp_ref[1, 0:N, :]
    z = dv * acc + (dv * dv) * hprev_ref[...] + b_ref[...]
    m = jnp.mean(z, axis=0, keepdims=True)
    v = jnp.mean((z - m) ** 2, axis=0, keepdims=True)
    zn = (z - m) * lax.rsqrt(v + 1e-5) * g_ref[...] + be_ref[...]
    r = jnp.maximum(zn, 0.0)
    h = jnp.dot(r, w_ref[...], preferred_element_type=f32)
    h_ref[...] = h
    hsp_ref[0:N, :] = dv * h
    hsp_ref[N:NPAD, :] = jnp.zeros((NPAD - N, D), f32)


def _tc_b(accp, hprev, dp, b, g, be, w):
    return pl.pallas_call(
        _tc_b_body,
        out_shape=[
            jax.ShapeDtypeStruct((N, D), f32),
            jax.ShapeDtypeStruct((NPAD, D), f32),
        ],
    )(accp, hprev, dp, b, g, be, w)


def _tc_d_body(accp_ref, hprev_ref, dp_ref, b_ref, w0t_ref, w0b_ref, eb0_ref,
               ref_ref, ap_ref, bp_ref):
    dv = _dinv_from_dp(dp_ref[...])
    acc = accp_ref[0, 0:N, :] + accp_ref[1, 0:N, :]
    refined = dv * acc + (dv * dv) * hprev_ref[...] + b_ref[...]
    ref_ref[...] = refined
    a = jnp.dot(refined, w0t_ref[...], preferred_element_type=f32) + eb0_ref[...]
    b = jnp.dot(refined, w0b_ref[...], preferred_element_type=f32)
    ap_ref[0:N, :] = a
    ap_ref[N:NPAD, :] = jnp.zeros((NPAD - N, D), f32)
    bp_ref[0:N, :] = b
    bp_ref[N:NPAD, :] = jnp.zeros((NPAD - N, D), f32)


def _tc_d(accp, hprev, dp, b, w0t, w0b, eb0):
    return pl.pallas_call(
        _tc_d_body,
        out_shape=[
            jax.ShapeDtypeStruct((N, D), f32),
            jax.ShapeDtypeStruct((NPAD, D), f32),
            jax.ShapeDtypeStruct((NPAD, D), f32),
        ],
    )(accp, hprev, dp, b, w0t, w0b, eb0)


def _tc_e_body(p_ref, m_ref, eb1_ref, ew_ref):
    z = jnp.dot(p_ref[...], m_ref[...], preferred_element_type=f32) + eb1_ref[...]
    ew_ref[...] = 1.0 / (1.0 + jnp.exp(-z))


def _tc_e(pflat, m, eb1):
    p2 = pflat.reshape(EPAD * 16 // 128, 128)
    out = pl.pallas_call(
        _tc_e_body,
        out_shape=jax.ShapeDtypeStruct((EPAD * 16 // 128, 8), f32),
    )(p2, m, eb1)
    return out.reshape(EPAD)


def _jnp_deg(dstp, zeros16, ones16):
    dst = dstp.reshape(-1)
    deg = jnp.zeros((NPAD,), f32).at[dst].add(1.0)
    return jnp.zeros((2, NPAD, 16), f32).at[0].set(deg[:, None])


def _jnp_conv(hs, srcp, dstp, zeros128):
    src = srcp.reshape(-1)
    dst = dstp.reshape(-1)
    acc = jnp.zeros((NPAD, D), f32).at[dst].add(hs[src])
    return jnp.stack([acc, jnp.zeros_like(acc)])


def _jnp_edge(ap, bp, srcp, dstp, w1):
    src = srcp.reshape(-1)
    dst = dstp.reshape(-1)
    r = jnp.maximum(ap[src] + bp[dst], 0.0) * w1
    return r.reshape(EPAD, 8, 16).sum(axis=1)


def kernel(x, edge_index, W0, b0, W1, b1, W2, b2, g0, be0, g1, be1,
           eW0, eb0, eW1, eb1):
    pad = jnp.full((EPAD - E,), N, i32)
    srcp = jnp.concatenate([edge_index[0], pad]).reshape(EPAD // KB, KB)
    dstp = jnp.concatenate([edge_index[1], pad]).reshape(EPAD // KB, KB)

    zeros128 = jnp.zeros((NPAD, D), f32)
    ones128 = jnp.ones((KB, D), f32)
    gm = (jnp.arange(128, dtype=i32)[:, None] // 16
          == jnp.arange(8, dtype=i32)[None, :]).astype(f32)
    w1 = eW1[:, 0]
    eW0t = eW0[:D]
    eW0b = eW0[D:]

    dp = _sc_deg(dstp, zeros128, ones128)
    h0, hs0 = _tc_a(x, W0, dp)
    acc0 = _sc_conv(hs0, srcp, dstp, zeros128)
    h1, hs1 = _tc_b(acc0, h0, dp, b0, g0, be0, W1)
    acc1 = _sc_conv(hs1, srcp, dstp, zeros128)
    h2, hs2 = _tc_b(acc1, h1, dp, b1, g1, be1, W2)
    acc2 = _sc_conv(hs2, srcp, dstp, zeros128)
    refined, ap, bp = _tc_d(acc2, h2, dp, b2, eW0t, eW0b, eb0)
    p = _sc_edge(ap, bp, srcp, dstp, w1)
    ewp = _tc_e(p, gm, eb1)
    return refined, ewp[:E]


# speedup vs baseline: 1.1112x; 1.1112x over previous
import functools

import jax
import jax.numpy as jnp
from jax import lax
from jax.experimental import pallas as pl
from jax.experimental.pallas import tpu as pltpu
from jax.experimental.pallas import tpu_sc as plsc

N = 10000
E = 320000
D = 128

NC = 2
NS = 16
NW = NC * NS

NPAD = 10112
RPT = NPAD // NS
EPAD = 327680
EW = EPAD // NW
KB = 128
CH = EW // KB

_mesh = plsc.VectorSubcoreMesh(core_axis_name="c", subcore_axis_name="s")

f32 = jnp.float32
i32 = jnp.int32


def _wid():
    return lax.axis_index("s") * NC + lax.axis_index("c")


@functools.partial(
    pl.kernel,
    out_type=jax.ShapeDtypeStruct((NC, NPAD, D), f32),
    mesh=_mesh,
    scratch_types=[
        pltpu.VMEM((CH, KB), i32),
        pltpu.VMEM((KB, D), f32),
        pltpu.VMEM_SHARED((NPAD, D), f32),
    ],
)
def _sc_deg(dst_hbm, zeros16_hbm, ones16_hbm, out_hbm, idxd, ones_v, shared):
    c = lax.axis_index("c")
    s = lax.axis_index("s")
    w = _wid()
    pltpu.sync_copy(zeros16_hbm.at[pl.ds(s * RPT, RPT)],
                    shared.at[pl.ds(s * RPT, RPT)])
    pltpu.sync_copy(ones16_hbm, ones_v)
    pltpu.sync_copy(dst_hbm.at[pl.ds(w * CH, CH)], idxd)
    plsc.subcore_barrier()

    def body(r, _):
        pltpu.sync_copy(ones_v, shared.at[idxd.at[r]], add=True)
        return 0

    lax.fori_loop(0, CH, body, 0)
    plsc.subcore_barrier()
    pltpu.sync_copy(shared.at[pl.ds(s * RPT, RPT)],
                    out_hbm.at[c, pl.ds(s * RPT, RPT)])


TCH = EPAD // KB
CONV_F = 136
CONV_S = TCH // NS - CONV_F
EDGE_F = 120
EDGE_S = TCH // NS - EDGE_F


@functools.partial(
    pl.kernel,
    out_type=jax.ShapeDtypeStruct((NC, NPAD, D), f32),
    mesh=_mesh,
    scratch_types=[
        pltpu.VMEM((3, KB), i32),
        pltpu.VMEM((3, KB), i32),
        pltpu.VMEM((KB, D), f32),
        pltpu.VMEM((KB, D), f32),
        pltpu.VMEM((KB, D), f32),
        pltpu.VMEM_SHARED((NPAD, D), f32),
        pltpu.SemaphoreType.DMA,
        pltpu.SemaphoreType.DMA,
        pltpu.SemaphoreType.DMA,
        pltpu.SemaphoreType.DMA,
        pltpu.SemaphoreType.DMA,
        pltpu.SemaphoreType.DMA,
        pltpu.SemaphoreType.DMA,
    ],
)
def _sc_conv(hs_hbm, src_hbm, dst_hbm, zeros_hbm, out_hbm,
             sring, dring, rows0, rows1, rows2, shared,
             gsem, ss0, ss1, ss2, sd0, sd1, sd2):
    c = lax.axis_index("c")
    s = lax.axis_index("s")
    myn = jnp.where(c == 1, CONV_F, CONV_S)
    base = jnp.where(c == 1, s * CONV_F, NS * CONV_F + s * CONV_S)
    pltpu.sync_copy(zeros_hbm.at[pl.ds(s * RPT, RPT)],
                    shared.at[pl.ds(s * RPT, RPT)])
    plsc.subcore_barrier()

    allrows = (rows0, rows1, rows2)
    sss = (ss0, ss1, ss2)
    sds = (sd0, sd1, sd2)
    for j in range(3):
        pltpu.async_copy(src_hbm.at[base + j], sring.at[j], sss[j])
        pltpu.async_copy(dst_hbm.at[base + j], dring.at[j], sds[j])
    for j in range(3):
        pltpu.make_async_copy(src_hbm.at[base], sring.at[j], sss[j]).wait()
        pltpu.async_copy(hs_hbm.at[sring.at[j]], allrows[j], gsem)

    def half(r, rows, sslot, dslot, ssem, dsem):
        r3 = r + 3

        @pl.when(r < myn)
        def _():
            pltpu.make_async_copy(hs_hbm.at[sslot], rows, gsem).wait()

            @pl.when(r3 < myn)
            def _():
                pltpu.async_copy(src_hbm.at[base + r3], sslot, ssem)

            pltpu.make_async_copy(dst_hbm.at[base], dslot, dsem).wait()
            pltpu.sync_copy(rows, shared.at[dslot], add=True)

            @pl.when(r3 < myn)
            def _():
                pltpu.async_copy(dst_hbm.at[base + r3], dslot, dsem)
                pltpu.make_async_copy(src_hbm.at[base], sslot, ssem).wait()
                pltpu.async_copy(hs_hbm.at[sslot], rows, gsem)

    def body(i, _):
        for j in range(3):
            half(3 * i + j, allrows[j], sring.at[j], dring.at[j],
                 sss[j], sds[j])
        return 0

    lax.fori_loop(0, (myn + 2) // 3, body, 0)
    plsc.subcore_barrier()
    pltpu.sync_copy(shared.at[pl.ds(s * RPT, RPT)],
                    out_hbm.at[c, pl.ds(s * RPT, RPT)])


_G = 8


@functools.partial(
    pl.kernel,
    out_type=jax.ShapeDtypeStruct((EPAD * 16,), f32),
    mesh=_mesh,
    scratch_types=[
        pltpu.VMEM((2, KB), i32),
        pltpu.VMEM((2, KB), i32),
        pltpu.VMEM((KB, D), f32),
        pltpu.VMEM((KB, D), f32),
        pltpu.VMEM((KB, D), f32),
        pltpu.VMEM((KB, D), f32),
        pltpu.VMEM((_G * KB * 16,), f32),
        pltpu.VMEM((D,), f32),
        pltpu.SemaphoreType.DMA,
        pltpu.SemaphoreType.DMA,
        pltpu.SemaphoreType.DMA,
        pltpu.SemaphoreType.DMA,
    ],
)
def _sc_edge(a_hbm, b_hbm, row_hbm, col_hbm, w1_hbm, out_hbm,
             rring, cring, a0, b0, a1, b1, pbuf, w1_v, sema, semb, sr0, sr1):
    c = lax.axis_index("c")
    s = lax.axis_index("s")
    myn = jnp.where(c == 1, EDGE_F, EDGE_S)
    base = jnp.where(c == 1, s * EDGE_F, NS * EDGE_F + s * EDGE_S)
    pltpu.sync_copy(w1_hbm, w1_v)
    wv = [w1_v[pl.ds(16 * k, 16)] for k in range(D // 16)]

    def load_idx(r, slot, sem):
        pltpu.async_copy(row_hbm.at[base + r], rring.at[slot], sem)
        pltpu.async_copy(col_hbm.at[base + r], cring.at[slot], sem)

    def wait_idx(slot, sem):
        pltpu.make_async_copy(row_hbm.at[base], rring.at[slot], sem).wait()
        pltpu.make_async_copy(col_hbm.at[base], cring.at[slot], sem).wait()

    load_idx(0, 0, sr0)
    load_idx(1, 1, sr1)
    wait_idx(0, sr0)
    pltpu.async_copy(a_hbm.at[rring.at[0]], a0, sema)
    pltpu.async_copy(b_hbm.at[cring.at[0]], b0, semb)
    wait_idx(1, sr1)
    pltpu.async_copy(a_hbm.at[rring.at[1]], a1, sema)
    pltpu.async_copy(b_hbm.at[cring.at[1]], b1, semb)

    def outer(g, _):
        rbase = g * _G
        for t in range(_G):
            r = rbase + t
            r2 = r + 2
            slot = t % 2
            cur_a, cur_b = (a0, b0) if slot == 0 else (a1, b1)
            ssem = sr0 if slot == 0 else sr1
            pltpu.make_async_copy(a_hbm.at[rring.at[slot]], cur_a, sema).wait()
            pltpu.make_async_copy(b_hbm.at[cring.at[slot]], cur_b, semb).wait()

            @pl.when(r2 < myn)
            def _():
                load_idx(r2, slot, ssem)

            def edge(e, _):
                acc = jnp.zeros((16,), f32)
                for k in range(D // 16):
                    va = cur_a[e, pl.ds(16 * k, 16)]
                    vb = cur_b[e, pl.ds(16 * k, 16)]
                    acc = acc + jnp.maximum(va + vb, 0.0) * wv[k]
                pbuf[pl.ds((t * KB + e) * 16, 16)] = acc
                return 0

            lax.fori_loop(0, KB, edge, 0)

            @pl.when(r2 < myn)
            def _():
                wait_idx(slot, ssem)
                pltpu.async_copy(a_hbm.at[rring.at[slot]], cur_a, sema)
                pltpu.async_copy(b_hbm.at[cring.at[slot]], cur_b, semb)

        pltpu.sync_copy(
            pbuf, out_hbm.at[pl.ds((base + rbase) * KB * 16, _G * KB * 16)])
        return 0

    lax.fori_loop(0, myn // _G, outer, 0)


def _dinv_from_dp(dp):
    deg = dp[0, :N, 0:1] + dp[1, :N, 0:1] + 1.0
    return lax.rsqrt(deg)


def _tc_a_body(x_ref, w_ref, dp_ref, h_ref, hsp_ref):
    dv = _dinv_from_dp(dp_ref[...])
    h = jnp.dot(x_ref[...], w_ref[...], preferred_element_type=f32)
    h_ref[...] = h
    hsp_ref[0:N, :] = dv * h
    hsp_ref[N:NPAD, :] = jnp.zeros((NPAD - N, D), f32)


def _tc_a(x, w0, dp):
    return pl.pallas_call(
        _tc_a_body,
        out_shape=[
            jax.ShapeDtypeStruct((N, D), f32),
            jax.ShapeDtypeStruct((NPAD, D), f32),
        ],
    )(x, w0, dp)


def _tc_b_body(accp_ref, hprev_ref, dp_ref, b_ref, g_ref, be_ref, w_ref,
               h_ref, hsp_ref):
    dv = _dinv_from_dp(dp_ref[...])
    acc = accp_ref[0, 0:N, :] + accp_ref[1, 0:N, :]
    z = dv * acc + (dv * dv) * hprev_ref[...] + b_ref[...]
    m = jnp.mean(z, axis=0, keepdims=True)
    v = jnp.mean((z - m) ** 2, axis=0, keepdims=True)
    zn = (z - m) * lax.rsqrt(v + 1e-5) * g_ref[...] + be_ref[...]
    r = jnp.maximum(zn, 0.0)
    h = jnp.dot(r, w_ref[...], preferred_element_type=f32)
    h_ref[...] = h
    hsp_ref[0:N, :] = dv * h
    hsp_ref[N:NPAD, :] = jnp.zeros((NPAD - N, D), f32)


def _tc_b(accp, hprev, dp, b, g, be, w):
    return pl.pallas_call(
        _tc_b_body,
        out_shape=[
            jax.ShapeDtypeStruct((N, D), f32),
            jax.ShapeDtypeStruct((NPAD, D), f32),
        ],
    )(accp, hprev, dp, b, g, be, w)


def _tc_d_body(accp_ref, hprev_ref, dp_ref, b_ref, w0t_ref, w0b_ref, eb0_ref,
               ref_ref, ap_ref, bp_ref):
    dv = _dinv_from_dp(dp_ref[...])
    acc = accp_ref[0, 0:N, :] + accp_ref[1, 0:N, :]
    refined = dv * acc + (dv * dv) * hprev_ref[...] + b_ref[...]
    ref_ref[...] = refined
    a = jnp.dot(refined, w0t_ref[...], preferred_element_type=f32) + eb0_ref[...]
    b = jnp.dot(refined, w0b_ref[...], preferred_element_type=f32)
    ap_ref[0:N, :] = a
    ap_ref[N:NPAD, :] = jnp.zeros((NPAD - N, D), f32)
    bp_ref[0:N, :] = b
    bp_ref[N:NPAD, :] = jnp.zeros((NPAD - N, D), f32)


def _tc_d(accp, hprev, dp, b, w0t, w0b, eb0):
    return pl.pallas_call(
        _tc_d_body,
        out_shape=[
            jax.ShapeDtypeStruct((N, D), f32),
            jax.ShapeDtypeStruct((NPAD, D), f32),
            jax.ShapeDtypeStruct((NPAD, D), f32),
        ],
    )(accp, hprev, dp, b, w0t, w0b, eb0)


def _tc_e_body(p_ref, m_ref, eb1_ref, ew_ref):
    z = jnp.dot(p_ref[...], m_ref[...], preferred_element_type=f32) + eb1_ref[...]
    ew_ref[...] = 1.0 / (1.0 + jnp.exp(-z))


def _tc_e(pflat, m, eb1):
    p2 = pflat.reshape(EPAD * 16 // 128, 128)
    out = pl.pallas_call(
        _tc_e_body,
        out_shape=jax.ShapeDtypeStruct((EPAD * 16 // 128, 8), f32),
    )(p2, m, eb1)
    return out.reshape(EPAD)


def _jnp_deg(dstp, zeros16, ones16):
    dst = dstp.reshape(-1)
    deg = jnp.zeros((NPAD,), f32).at[dst].add(1.0)
    return jnp.zeros((2, NPAD, 16), f32).at[0].set(deg[:, None])


def _jnp_conv(hs, srcp, dstp, zeros128):
    src = srcp.reshape(-1)
    dst = dstp.reshape(-1)
    acc = jnp.zeros((NPAD, D), f32).at[dst].add(hs[src])
    return jnp.stack([acc, jnp.zeros_like(acc)])


def _jnp_edge(ap, bp, srcp, dstp, w1):
    src = srcp.reshape(-1)
    dst = dstp.reshape(-1)
    r = jnp.maximum(ap[src] + bp[dst], 0.0) * w1
    return r.reshape(EPAD, 8, 16).sum(axis=1)


def kernel(x, edge_index, W0, b0, W1, b1, W2, b2, g0, be0, g1, be1,
           eW0, eb0, eW1, eb1):
    pad = jnp.full((EPAD - E,), N, i32)
    srcp = jnp.concatenate([edge_index[0], pad]).reshape(EPAD // KB, KB)
    dstp = jnp.concatenate([edge_index[1], pad]).reshape(EPAD // KB, KB)

    zeros128 = jnp.zeros((NPAD, D), f32)
    ones128 = jnp.ones((KB, D), f32)
    gm = (jnp.arange(128, dtype=i32)[:, None] // 16
          == jnp.arange(8, dtype=i32)[None, :]).astype(f32)
    w1 = eW1[:, 0]
    eW0t = eW0[:D]
    eW0b = eW0[D:]

    dp = _sc_deg(dstp, zeros128, ones128)
    h0, hs0 = _tc_a(x, W0, dp)
    acc0 = _sc_conv(hs0, srcp, dstp, zeros128)
    h1, hs1 = _tc_b(acc0, h0, dp, b0, g0, be0, W1)
    acc1 = _sc_conv(hs1, srcp, dstp, zeros128)
    h2, hs2 = _tc_b(acc1, h1, dp, b1, g1, be1, W2)
    acc2 = _sc_conv(hs2, srcp, dstp, zeros128)
    refined, ap, bp = _tc_d(acc2, h2, dp, b2, eW0t, eW0b, eb0)
    p = _sc_edge(ap, bp, srcp, dstp, w1)
    ewp = _tc_e(p, gm, eb1)
    return refined, ewp[:E]


# speedup vs baseline: 1.3410x; 1.2069x over previous
import functools

import jax
import jax.numpy as jnp
from jax import lax
from jax.experimental import pallas as pl
from jax.experimental.pallas import tpu as pltpu
from jax.experimental.pallas import tpu_sc as plsc

N = 10000
E = 320000
D = 128

NC = 2
NS = 16
NW = NC * NS

NPAD = 10112
RPT = NPAD // NS
EPAD = 327680
EW = EPAD // NW
KB = 128
CH = EW // KB

_mesh = plsc.VectorSubcoreMesh(core_axis_name="c", subcore_axis_name="s")

f32 = jnp.float32
i32 = jnp.int32


def _wid():
    return lax.axis_index("s") * NC + lax.axis_index("c")


@functools.partial(
    pl.kernel,
    out_type=jax.ShapeDtypeStruct((NC, NPAD, D), f32),
    mesh=_mesh,
    scratch_types=[
        pltpu.VMEM((CH, KB), i32),
        pltpu.VMEM((KB, D), f32),
        pltpu.VMEM_SHARED((NPAD, D), f32),
    ],
)
def _sc_deg(dst_hbm, zeros16_hbm, ones16_hbm, out_hbm, idxd, ones_v, shared):
    c = lax.axis_index("c")
    s = lax.axis_index("s")
    w = _wid()
    pltpu.sync_copy(zeros16_hbm.at[pl.ds(s * RPT, RPT)],
                    shared.at[pl.ds(s * RPT, RPT)])
    pltpu.sync_copy(ones16_hbm, ones_v)
    pltpu.sync_copy(dst_hbm.at[pl.ds(w * CH, CH)], idxd)
    plsc.subcore_barrier()

    def body(r, _):
        pltpu.sync_copy(ones_v, shared.at[idxd.at[r]], add=True)
        return 0

    lax.fori_loop(0, CH, body, 0)
    plsc.subcore_barrier()
    pltpu.sync_copy(shared.at[pl.ds(s * RPT, RPT)],
                    out_hbm.at[c, pl.ds(s * RPT, RPT)])


TCH = EPAD // KB
CONV_F = 152
CONV_S = TCH // NS - CONV_F
EDGE_F = 144
EDGE_S = TCH // NS - EDGE_F


@functools.partial(
    pl.kernel,
    out_type=jax.ShapeDtypeStruct((NC, NPAD, D), f32),
    mesh=_mesh,
    scratch_types=[
        pltpu.VMEM((3, KB), i32),
        pltpu.VMEM((3, KB), i32),
        pltpu.VMEM((KB, D), f32),
        pltpu.VMEM((KB, D), f32),
        pltpu.VMEM((KB, D), f32),
        pltpu.VMEM_SHARED((NPAD, D), f32),
        pltpu.SemaphoreType.DMA,
        pltpu.SemaphoreType.DMA,
        pltpu.SemaphoreType.DMA,
        pltpu.SemaphoreType.DMA,
        pltpu.SemaphoreType.DMA,
        pltpu.SemaphoreType.DMA,
        pltpu.SemaphoreType.DMA,
    ],
)
def _sc_conv(hs_hbm, src_hbm, dst_hbm, zeros_hbm, out_hbm,
             sring, dring, rows0, rows1, rows2, shared,
             gsem, ss0, ss1, ss2, sd0, sd1, sd2):
    c = lax.axis_index("c")
    s = lax.axis_index("s")
    myn = jnp.where(c == 1, CONV_F, CONV_S)
    base = jnp.where(c == 1, s * CONV_F, NS * CONV_F + s * CONV_S)
    pltpu.sync_copy(zeros_hbm.at[pl.ds(s * RPT, RPT)],
                    shared.at[pl.ds(s * RPT, RPT)])
    plsc.subcore_barrier()

    allrows = (rows0, rows1, rows2)
    sss = (ss0, ss1, ss2)
    sds = (sd0, sd1, sd2)
    for j in range(3):
        pltpu.async_copy(src_hbm.at[base + j], sring.at[j], sss[j])
        pltpu.async_copy(dst_hbm.at[base + j], dring.at[j], sds[j])
    for j in range(3):
        pltpu.make_async_copy(src_hbm.at[base], sring.at[j], sss[j]).wait()
        pltpu.async_copy(hs_hbm.at[sring.at[j]], allrows[j], gsem)

    def half(r, rows, sslot, dslot, ssem, dsem):
        r3 = r + 3

        @pl.when(r < myn)
        def _():
            pltpu.make_async_copy(hs_hbm.at[sslot], rows, gsem).wait()

            @pl.when(r3 < myn)
            def _():
                pltpu.async_copy(src_hbm.at[base + r3], sslot, ssem)

            pltpu.make_async_copy(dst_hbm.at[base], dslot, dsem).wait()
            pltpu.sync_copy(rows, shared.at[dslot], add=True)

            @pl.when(r3 < myn)
            def _():
                pltpu.async_copy(dst_hbm.at[base + r3], dslot, dsem)
                pltpu.make_async_copy(src_hbm.at[base], sslot, ssem).wait()
                pltpu.async_copy(hs_hbm.at[sslot], rows, gsem)

    def body(i, _):
        for j in range(3):
            half(3 * i + j, allrows[j], sring.at[j], dring.at[j],
                 sss[j], sds[j])
        return 0

    lax.fori_loop(0, (myn + 2) // 3, body, 0)
    plsc.subcore_barrier()
    pltpu.sync_copy(shared.at[pl.ds(s * RPT, RPT)],
                    out_hbm.at[c, pl.ds(s * RPT, RPT)])


_G = 8


@functools.partial(
    pl.kernel,
    out_type=jax.ShapeDtypeStruct((EPAD * 16,), f32),
    mesh=_mesh,
    scratch_types=[
        pltpu.VMEM((2, KB), i32),
        pltpu.VMEM((2, KB), i32),
        pltpu.VMEM((KB, D), f32),
        pltpu.VMEM((KB, D), f32),
        pltpu.VMEM((KB, D), f32),
        pltpu.VMEM((KB, D), f32),
        pltpu.VMEM((_G * KB * 16,), f32),
        pltpu.VMEM((D,), f32),
        pltpu.SemaphoreType.DMA,
        pltpu.SemaphoreType.DMA,
        pltpu.SemaphoreType.DMA,
        pltpu.SemaphoreType.DMA,
    ],
)
def _sc_edge(a_hbm, b_hbm, row_hbm, col_hbm, w1_hbm, out_hbm,
             rring, cring, a0, b0, a1, b1, pbuf, w1_v, sema, semb, sr0, sr1):
    c = lax.axis_index("c")
    s = lax.axis_index("s")
    myn = jnp.where(c == 1, EDGE_F, EDGE_S)
    base = jnp.where(c == 1, s * EDGE_F, NS * EDGE_F + s * EDGE_S)
    pltpu.sync_copy(w1_hbm, w1_v)
    wv = [w1_v[pl.ds(16 * k, 16)] for k in range(D // 16)]

    def load_idx(r, slot, sem):
        pltpu.async_copy(row_hbm.at[base + r], rring.at[slot], sem)
        pltpu.async_copy(col_hbm.at[base + r], cring.at[slot], sem)

    def wait_idx(slot, sem):
        pltpu.make_async_copy(row_hbm.at[base], rring.at[slot], sem).wait()
        pltpu.make_async_copy(col_hbm.at[base], cring.at[slot], sem).wait()

    load_idx(0, 0, sr0)
    load_idx(1, 1, sr1)
    wait_idx(0, sr0)
    pltpu.async_copy(a_hbm.at[rring.at[0]], a0, sema)
    pltpu.async_copy(b_hbm.at[cring.at[0]], b0, semb)
    wait_idx(1, sr1)
    pltpu.async_copy(a_hbm.at[rring.at[1]], a1, sema)
    pltpu.async_copy(b_hbm.at[cring.at[1]], b1, semb)

    def outer(g, _):
        rbase = g * _G
        for t in range(_G):
            r = rbase + t
            r2 = r + 2
            slot = t % 2
            cur_a, cur_b = (a0, b0) if slot == 0 else (a1, b1)
            ssem = sr0 if slot == 0 else sr1
            pltpu.make_async_copy(a_hbm.at[rring.at[slot]], cur_a, sema).wait()
            pltpu.make_async_copy(b_hbm.at[cring.at[slot]], cur_b, semb).wait()

            @pl.when(r2 < myn)
            def _():
                load_idx(r2, slot, ssem)

            def edge(e, _):
                acc = jnp.zeros((16,), f32)
                for k in range(D // 16):
                    va = cur_a[e, pl.ds(16 * k, 16)]
                    vb = cur_b[e, pl.ds(16 * k, 16)]
                    acc = acc + jnp.maximum(va + vb, 0.0) * wv[k]
                pbuf[pl.ds((t * KB + e) * 16, 16)] = acc
                return 0

            lax.fori_loop(0, KB, edge, 0)

            @pl.when(r2 < myn)
            def _():
                wait_idx(slot, ssem)
                pltpu.async_copy(a_hbm.at[rring.at[slot]], cur_a, sema)
                pltpu.async_copy(b_hbm.at[cring.at[slot]], cur_b, semb)

        pltpu.sync_copy(
            pbuf, out_hbm.at[pl.ds((base + rbase) * KB * 16, _G * KB * 16)])
        return 0

    lax.fori_loop(0, myn // _G, outer, 0)


def _dinv_from_dp(dp):
    deg = dp[0, :N, 0:1] + dp[1, :N, 0:1] + 1.0
    return lax.rsqrt(deg)


def _tc_a_body(x_ref, w_ref, dp_ref, h_ref, hsp_ref):
    dv = _dinv_from_dp(dp_ref[...])
    h = jnp.dot(x_ref[...], w_ref[...], preferred_element_type=f32)
    h_ref[...] = h
    hsp_ref[0:N, :] = dv * h
    hsp_ref[N:NPAD, :] = jnp.zeros((NPAD - N, D), f32)


def _tc_a(x, w0, dp):
    return pl.pallas_call(
        _tc_a_body,
        out_shape=[
            jax.ShapeDtypeStruct((N, D), f32),
            jax.ShapeDtypeStruct((NPAD, D), f32),
        ],
    )(x, w0, dp)


def _tc_b_body(accp_ref, hprev_ref, dp_ref, b_ref, g_ref, be_ref, w_ref,
               h_ref, hsp_ref):
    dv = _dinv_from_dp(dp_ref[...])
    acc = accp_ref[0, 0:N, :] + accp_ref[1, 0:N, :]
    z = dv * acc + (dv * dv) * hprev_ref[...] + b_ref[...]
    m = jnp.mean(z, axis=0, keepdims=True)
    v = jnp.mean((z - m) ** 2, axis=0, keepdims=True)
    zn = (z - m) * lax.rsqrt(v + 1e-5) * g_ref[...] + be_ref[...]
    r = jnp.maximum(zn, 0.0)
    h = jnp.dot(r, w_ref[...], preferred_element_type=f32)
    h_ref[...] = h
    hsp_ref[0:N, :] = dv * h
    hsp_ref[N:NPAD, :] = jnp.zeros((NPAD - N, D), f32)


def _tc_b(accp, hprev, dp, b, g, be, w):
    return pl.pallas_call(
        _tc_b_body,
        out_shape=[
            jax.ShapeDtypeStruct((N, D), f32),
            jax.ShapeDtypeStruct((NPAD, D), f32),
        ],
    )(accp, hprev, dp, b, g, be, w)


def _tc_d_body(accp_ref, hprev_ref, dp_ref, b_ref, w0t_ref, w0b_ref, eb0_ref,
               ref_ref, ap_ref, bp_ref):
    dv = _dinv_from_dp(dp_ref[...])
    acc = accp_ref[0, 0:N, :] + accp_ref[1, 0:N, :]
    refined = dv * acc + (dv * dv) * hprev_ref[...] + b_ref[...]
    ref_ref[...] = refined
    a = jnp.dot(refined, w0t_ref[...], preferred_element_type=f32) + eb0_ref[...]
    b = jnp.dot(refined, w0b_ref[...], preferred_element_type=f32)
    ap_ref[0:N, :] = a
    ap_ref[N:NPAD, :] = jnp.zeros((NPAD - N, D), f32)
    bp_ref[0:N, :] = b
    bp_ref[N:NPAD, :] = jnp.zeros((NPAD - N, D), f32)


def _tc_d(accp, hprev, dp, b, w0t, w0b, eb0):
    return pl.pallas_call(
        _tc_d_body,
        out_shape=[
            jax.ShapeDtypeStruct((N, D), f32),
            jax.ShapeDtypeStruct((NPAD, D), f32),
            jax.ShapeDtypeStruct((NPAD, D), f32),
        ],
    )(accp, hprev, dp, b, w0t, w0b, eb0)


def _tc_e_body(p_ref, m_ref, eb1_ref, ew_ref):
    z = jnp.dot(p_ref[...], m_ref[...], preferred_element_type=f32) + eb1_ref[...]
    ew_ref[...] = 1.0 / (1.0 + jnp.exp(-z))


def _tc_e(pflat, m, eb1):
    p2 = pflat.reshape(EPAD * 16 // 128, 128)
    out = pl.pallas_call(
        _tc_e_body,
        out_shape=jax.ShapeDtypeStruct((EPAD * 16 // 128, 8), f32),
    )(p2, m, eb1)
    return out.reshape(EPAD)


def _jnp_deg(dstp, zeros16, ones16):
    dst = dstp.reshape(-1)
    deg = jnp.zeros((NPAD,), f32).at[dst].add(1.0)
    return jnp.zeros((2, NPAD, 16), f32).at[0].set(deg[:, None])


def _jnp_conv(hs, srcp, dstp, zeros128):
    src = srcp.reshape(-1)
    dst = dstp.reshape(-1)
    acc = jnp.zeros((NPAD, D), f32).at[dst].add(hs[src])
    return jnp.stack([acc, jnp.zeros_like(acc)])


def _jnp_edge(ap, bp, srcp, dstp, w1):
    src = srcp.reshape(-1)
    dst = dstp.reshape(-1)
    r = jnp.maximum(ap[src] + bp[dst], 0.0) * w1
    return r.reshape(EPAD, 8, 16).sum(axis=1)


def kernel(x, edge_index, W0, b0, W1, b1, W2, b2, g0, be0, g1, be1,
           eW0, eb0, eW1, eb1):
    pad = jnp.full((EPAD - E,), N, i32)
    srcp = jnp.concatenate([edge_index[0], pad]).reshape(EPAD // KB, KB)
    dstp = jnp.concatenate([edge_index[1], pad]).reshape(EPAD // KB, KB)

    zeros128 = jnp.zeros((NPAD, D), f32)
    ones128 = jnp.ones((KB, D), f32)
    gm = (jnp.arange(128, dtype=i32)[:, None] // 16
          == jnp.arange(8, dtype=i32)[None, :]).astype(f32)
    w1 = eW1[:, 0]
    eW0t = eW0[:D]
    eW0b = eW0[D:]

    dp = _sc_deg(dstp, zeros128, ones128)
    h0, hs0 = _tc_a(x, W0, dp)
    acc0 = _sc_conv(hs0, srcp, dstp, zeros128)
    h1, hs1 = _tc_b(acc0, h0, dp, b0, g0, be0, W1)
    acc1 = _sc_conv(hs1, srcp, dstp, zeros128)
    h2, hs2 = _tc_b(acc1, h1, dp, b1, g1, be1, W2)
    acc2 = _sc_conv(hs2, srcp, dstp, zeros128)
    refined, ap, bp = _tc_d(acc2, h2, dp, b2, eW0t, eW0b, eb0)
    p = _sc_edge(ap, bp, srcp, dstp, w1)
    ewp = _tc_e(p, gm, eb1)
    return refined, ewp[:E]
